# baseline (speedup 1.0000x reference)
import functools

import jax
import jax.numpy as jnp
from jax import lax
from jax.experimental import pallas as pl
from jax.experimental.pallas import tpu as pltpu
from jax.experimental.pallas import tpu_sc as plsc

NC = 2
NS = 16
CH = 128
BLK = 32


def _make_seg_kernel(n_acc, n_src_rows, e_pad, d, with_counts):
  rpt = n_acc // NS
  ept = e_pad // NS
  nch = ept // CH
  nblk = nch // BLK

  out_type = [jax.ShapeDtypeStruct((n_acc, d), jnp.float32)] * (
      4 if with_counts else 2)
  scratch = [
      pltpu.VMEM_SHARED((n_acc, d), jnp.float32),
      pltpu.VMEM((BLK + 1, CH), jnp.int32),
      pltpu.VMEM((BLK, CH), jnp.int32),
      pltpu.VMEM((2, CH, d), jnp.float32),
      pltpu.SemaphoreType.DMA,
      pltpu.SemaphoreType.DMA,
      pltpu.SemaphoreType.DMA,
      pltpu.SemaphoreType.DMA,
  ]

  mesh = plsc.VectorSubcoreMesh(core_axis_name="c", subcore_axis_name="s",
                                num_cores=NC, num_subcores=NS)

  def body(*refs):
    if with_counts:
      (x0, x1, si0, di0, si1, di1,
       sum0, sum1, cnt0, cnt1, acc, sall, dall, rows, g0, g1, t0, t1) = refs
    else:
      (x0, x1, si0, di0, si1, di1,
       sum0, sum1, acc, sall, dall, rows, g0, g1, t0, t1) = refs
    cid = lax.axis_index("c")
    sid = lax.axis_index("s")
    r0 = sid * rpt

    def fill(buf, value, dtype):
      v = jnp.full((16,), value, dtype)

      def fr(r, carry):
        for k in range(d // 16):
          buf[r, pl.ds(k * 16, 16)] = v
        return carry
      lax.fori_loop(0, CH, fr, 0)

    def zero_acc():
      fill(rows.at[1], 0.0, jnp.float32)
      for j in range(rpt // CH):
        pltpu.sync_copy(rows.at[1], acc.at[pl.ds(r0 + j * CH, CH)])

    def writeout(o_ref):
      for j in range(rpt // CH):
        pltpu.sync_copy(acc.at[pl.ds(r0 + j * CH, CH)], rows.at[0])
        pltpu.sync_copy(rows.at[0], o_ref.at[pl.ds(r0 + j * CH, CH)])

    def wait_gather(x_hbm, b, sem):
      pltpu.make_async_copy(x_hbm.at[sall.at[0]], rows.at[b], sem).wait()

    def wait_scat(b, sem):
      pltpu.make_async_copy(rows.at[b], acc.at[dall.at[0]], sem).wait()

    zero_acc()
    plsc.subcore_barrier()

    def do_edges(x_hbm, s3, d3):
      zvec = jnp.zeros((16,), jnp.int32)
      for k in range(CH // 16):
        sall[BLK, pl.ds(k * 16, 16)] = zvec

      def block(blk, carry):
        pltpu.sync_copy(s3.at[sid, pl.ds(blk * BLK, BLK)],
                        sall.at[pl.ds(0, BLK)])
        pltpu.sync_copy(d3.at[sid, pl.ds(blk * BLK, BLK)], dall)
        pltpu.async_copy(x_hbm.at[sall.at[0]], rows.at[0], g0)

        def pair(p, carry2):
          c0 = 2 * p
          wait_gather(x_hbm, 0, g0)
          pltpu.async_copy(rows.at[0], acc.at[dall.at[c0]], t0, add=True)
          pl.when(p > 0)(lambda: wait_scat(1, t1))
          pltpu.async_copy(x_hbm.at[sall.at[c0 + 1]], rows.at[1], g1)
          wait_gather(x_hbm, 1, g1)
          pltpu.async_copy(rows.at[1], acc.at[dall.at[c0 + 1]], t1, add=True)
          wait_scat(0, t0)
          pltpu.async_copy(x_hbm.at[sall.at[c0 + 2]], rows.at[0], g0)
          return carry2
        lax.fori_loop(0, BLK // 2, pair, 0)
        wait_gather(x_hbm, 0, g0)
        wait_scat(1, t1)
        return carry
      lax.fori_loop(0, nblk, block, 0)

    pl.when(cid == 0)(lambda: do_edges(x0, si0, di0))
    pl.when(cid == 1)(lambda: do_edges(x1, si1, di1))
    plsc.subcore_barrier()
    pl.when(cid == 0)(lambda: writeout(sum0))
    pl.when(cid == 1)(lambda: writeout(sum1))

    if with_counts:
      zero_acc()
      fill(rows.at[0], 1.0, jnp.float32)
      plsc.subcore_barrier()

      def do_counts(d3):
        def cblock(blk, carry):
          @pl.when(blk > 0)
          def _drain():
            wait_scat(0, t0)
            wait_scat(0, t1)
          pltpu.sync_copy(d3.at[sid, pl.ds(blk * BLK, BLK)], dall)

          def cpair(p, carry2):
            pl.when(p > 0)(lambda: wait_scat(0, t0))
            pltpu.async_copy(rows.at[0], acc.at[dall.at[2 * p]], t0,
                             add=True)
            pl.when(p > 0)(lambda: wait_scat(0, t1))
            pltpu.async_copy(rows.at[0], acc.at[dall.at[2 * p + 1]], t1,
                             add=True)
            return carry2
          lax.fori_loop(0, BLK // 2, cpair, 0)
          return carry
        lax.fori_loop(0, nblk, cblock, 0)
        wait_scat(0, t0)
        wait_scat(0, t1)

      pl.when(cid == 0)(lambda: do_counts(di0))
      pl.when(cid == 1)(lambda: do_counts(di1))
      plsc.subcore_barrier()
      pl.when(cid == 0)(lambda: writeout(cnt0))
      pl.when(cid == 1)(lambda: writeout(cnt1))

  return pl.kernel(body, out_type=out_type, mesh=mesh, scratch_types=scratch)


def _make_dense_kernel(n, n_acc, d, out_rows):
  def one(s_ref, c_ref, x_ref, wl_ref, bl_ref, wr_ref, g_ref, be_ref, o_ref):
    cnt = jnp.maximum(c_ref[0:n, 0:1], 1.0)
    mean = s_ref[0:n, :] / cnt
    h = jnp.dot(mean, wl_ref[...], preferred_element_type=jnp.float32)
    h = h + bl_ref[...]
    h = h + jnp.dot(x_ref[...], wr_ref[...], preferred_element_type=jnp.float32)
    mu = jnp.mean(h, axis=0, keepdims=True)
    var = jnp.mean(jnp.square(h - mu), axis=0, keepdims=True)
    y = (h - mu) * lax.rsqrt(var + 1e-5) * g_ref[...] + be_ref[...]
    y = jnp.where(y > 0, y, jnp.exp(jnp.minimum(y, 0.0)) - 1.0)
    o_ref[0:n, :] = y
    if out_rows > n:
      o_ref[n:out_rows, :] = jnp.zeros((out_rows - n, d), jnp.float32)

  def body(s0, c0, x0, wl0, bl0, wr0, g0, be0,
           s1, c1, x1, wl1, bl1, wr1, g1, be1, o0, o1):
    one(s0, c0, x0, wl0, bl0, wr0, g0, be0, o0)
    one(s1, c1, x1, wl1, bl1, wr1, g1, be1, o1)

  return pl.pallas_call(
      body,
      out_shape=[jax.ShapeDtypeStruct((out_rows, d), jnp.float32)] * 2,
  )


def kernel(x_user, x_item, edge_index_ui, edge_index_iu,
           Wl0_ui, bl0_ui, Wr0_ui, Wl0_iu, bl0_iu, Wr0_iu,
           g0_u, be0_u, g0_i, be0_i,
           Wl1_ui, bl1_ui, Wr1_ui, Wl1_iu, bl1_iu, Wr1_iu,
           g1_u, be1_u, g1_i, be1_i):
  n, d = x_user.shape
  e = edge_index_ui.shape[1]

  n_acc = -(-(n + 1) // (NS * CH)) * (NS * CH)
  n_src = n + 8
  e_pad = -(-e // (NS * CH * BLK)) * (NS * CH * BLK)
  nch = e_pad // (NS * CH)

  i32 = jnp.int32
  pad_idx = jnp.full((e_pad - e,), n, i32)
  r3 = lambda a: a.reshape(NS, nch, CH)
  s_ui = r3(jnp.concatenate([edge_index_ui[0].astype(i32), pad_idx]))
  d_ui = r3(jnp.concatenate([edge_index_ui[1].astype(i32), pad_idx]))
  s_iu = r3(jnp.concatenate([edge_index_iu[0].astype(i32), pad_idx]))
  d_iu = r3(jnp.concatenate([edge_index_iu[1].astype(i32), pad_idx]))

  zrow = jnp.zeros((n_src - n, d), jnp.float32)
  xu_pad = jnp.concatenate([x_user, zrow])
  xi_pad = jnp.concatenate([x_item, zrow])

  seg_c = _make_seg_kernel(n_acc, n_src, e_pad, d, with_counts=True)
  seg_n = _make_seg_kernel(n_acc, n_src, e_pad, d, with_counts=False)
  dense_pad = _make_dense_kernel(n, n_acc, d, n_src)
  dense_fin = _make_dense_kernel(n, n_acc, d, n)

  r2 = lambda v: v.reshape(1, d)

  sum_i0, sum_u0, cnt_i, cnt_u = seg_c(
      xu_pad, xi_pad, s_ui, d_ui, s_iu, d_iu)
  i1_pad, u1_pad = dense_pad(
      sum_i0, cnt_i, x_item, Wl0_ui, r2(bl0_ui), Wr0_ui, r2(g0_i), r2(be0_i),
      sum_u0, cnt_u, x_user, Wl0_iu, r2(bl0_iu), Wr0_iu, r2(g0_u), r2(be0_u))

  sum_i1, sum_u1 = seg_n(u1_pad, i1_pad, s_ui, d_ui, s_iu, d_iu)
  i2, u2 = dense_fin(
      sum_i1, cnt_i, i1_pad[0:n], Wl1_ui, r2(bl1_ui), Wr1_ui,
      r2(g1_i), r2(be1_i),
      sum_u1, cnt_u, u1_pad[0:n], Wl1_iu, r2(bl1_iu), Wr1_iu,
      r2(g1_u), r2(be1_u))

  return (x_user, x_item, u1_pad[0:n], i1_pad[0:n], u2, i2)


# speedup vs baseline: 1.0085x; 1.0085x over previous
import functools

import jax
import jax.numpy as jnp
from jax import lax
from jax.experimental import pallas as pl
from jax.experimental.pallas import tpu as pltpu
from jax.experimental.pallas import tpu_sc as plsc

NC = 2
NS = 16
CH = 128
BLK = 32


def _make_seg_kernel(n_acc, n_src_rows, e_pad, d, with_counts):
  rpt = n_acc // NS
  ept = e_pad // NS
  nch = ept // CH
  nblk = nch // BLK

  out_type = [jax.ShapeDtypeStruct((n_acc, d), jnp.float32)] * (
      4 if with_counts else 2)
  scratch = [
      pltpu.VMEM_SHARED((n_acc, d), jnp.float32),
      pltpu.VMEM((BLK + 1, CH), jnp.int32),
      pltpu.VMEM((BLK, CH), jnp.int32),
      pltpu.VMEM((2, CH, d), jnp.float32),
      pltpu.SemaphoreType.DMA,
      pltpu.SemaphoreType.DMA,
  ]

  mesh = plsc.VectorSubcoreMesh(core_axis_name="c", subcore_axis_name="s",
                                num_cores=NC, num_subcores=NS)

  def body(*refs):
    if with_counts:
      (x0, x1, si0, di0, si1, di1,
       sum0, sum1, cnt0, cnt1, acc, sall, dall, rows, g0, g1) = refs
    else:
      (x0, x1, si0, di0, si1, di1,
       sum0, sum1, acc, sall, dall, rows, g0, g1) = refs
    cid = lax.axis_index("c")
    sid = lax.axis_index("s")
    r0 = sid * rpt

    def fill(buf, value, dtype):
      v = jnp.full((16,), value, dtype)

      def fr(r, carry):
        for k in range(d // 16):
          buf[r, pl.ds(k * 16, 16)] = v
        return carry
      lax.fori_loop(0, CH, fr, 0)

    def zero_acc():
      fill(rows.at[1], 0.0, jnp.float32)
      for j in range(rpt // CH):
        pltpu.sync_copy(rows.at[1], acc.at[pl.ds(r0 + j * CH, CH)])

    def writeout(o_ref):
      for j in range(rpt // CH):
        pltpu.sync_copy(acc.at[pl.ds(r0 + j * CH, CH)], rows.at[0])
        pltpu.sync_copy(rows.at[0], o_ref.at[pl.ds(r0 + j * CH, CH)])

    def wait_gather(x_hbm, b, sem):
      pltpu.make_async_copy(x_hbm.at[sall.at[0]], rows.at[b], sem).wait()

    zero_acc()
    plsc.subcore_barrier()

    def do_edges(x_hbm, s3, d3):
      zvec = jnp.zeros((16,), jnp.int32)
      for k in range(CH // 16):
        sall[BLK, pl.ds(k * 16, 16)] = zvec

      def block(blk, carry):
        pltpu.sync_copy(s3.at[sid, pl.ds(blk * BLK, BLK)],
                        sall.at[pl.ds(0, BLK)])
        pltpu.sync_copy(d3.at[sid, pl.ds(blk * BLK, BLK)], dall)
        pltpu.async_copy(x_hbm.at[sall.at[0]], rows.at[0], g0)

        def pair(p, carry2):
          c0 = 2 * p
          wait_gather(x_hbm, 0, g0)
          pltpu.async_copy(x_hbm.at[sall.at[c0 + 1]], rows.at[1], g1)
          pltpu.sync_copy(rows.at[0], acc.at[dall.at[c0]], add=True)
          wait_gather(x_hbm, 1, g1)
          pltpu.async_copy(x_hbm.at[sall.at[c0 + 2]], rows.at[0], g0)
          pltpu.sync_copy(rows.at[1], acc.at[dall.at[c0 + 1]], add=True)
          return carry2
        lax.fori_loop(0, BLK // 2, pair, 0)
        wait_gather(x_hbm, 0, g0)
        return carry
      lax.fori_loop(0, nblk, block, 0)

    pl.when(cid == 0)(lambda: do_edges(x0, si0, di0))
    pl.when(cid == 1)(lambda: do_edges(x1, si1, di1))
    plsc.subcore_barrier()
    pl.when(cid == 0)(lambda: writeout(sum0))
    pl.when(cid == 1)(lambda: writeout(sum1))

    if with_counts:
      zero_acc()
      fill(rows.at[0], 1.0, jnp.float32)
      plsc.subcore_barrier()

      def do_counts(d3):
        def cblock(blk, carry):
          pltpu.sync_copy(d3.at[sid, pl.ds(blk * BLK, BLK)], dall)

          def cstep(c, carry2):
            pltpu.sync_copy(rows.at[0], acc.at[dall.at[c]], add=True)
            return carry2
          lax.fori_loop(0, BLK, cstep, 0)
          return carry
        lax.fori_loop(0, nblk, cblock, 0)

      pl.when(cid == 0)(lambda: do_counts(di0))
      pl.when(cid == 1)(lambda: do_counts(di1))
      plsc.subcore_barrier()
      pl.when(cid == 0)(lambda: writeout(cnt0))
      pl.when(cid == 1)(lambda: writeout(cnt1))

  return pl.kernel(body, out_type=out_type, mesh=mesh, scratch_types=scratch)


def _make_dense_kernel(n, n_acc, d, out_rows):
  def one(s_ref, c_ref, x_ref, wl_ref, bl_ref, wr_ref, g_ref, be_ref, o_ref):
    cnt = jnp.maximum(c_ref[0:n, 0:1], 1.0)
    mean = s_ref[0:n, :] / cnt
    h = jnp.dot(mean, wl_ref[...], preferred_element_type=jnp.float32)
    h = h + bl_ref[...]
    h = h + jnp.dot(x_ref[...], wr_ref[...], preferred_element_type=jnp.float32)
    mu = jnp.mean(h, axis=0, keepdims=True)
    var = jnp.mean(jnp.square(h - mu), axis=0, keepdims=True)
    y = (h - mu) * lax.rsqrt(var + 1e-5) * g_ref[...] + be_ref[...]
    y = jnp.where(y > 0, y, jnp.exp(jnp.minimum(y, 0.0)) - 1.0)
    o_ref[0:n, :] = y
    if out_rows > n:
      o_ref[n:out_rows, :] = jnp.zeros((out_rows - n, d), jnp.float32)

  def body(s0, c0, x0, wl0, bl0, wr0, g0, be0,
           s1, c1, x1, wl1, bl1, wr1, g1, be1, o0, o1):
    one(s0, c0, x0, wl0, bl0, wr0, g0, be0, o0)
    one(s1, c1, x1, wl1, bl1, wr1, g1, be1, o1)

  return pl.pallas_call(
      body,
      out_shape=[jax.ShapeDtypeStruct((out_rows, d), jnp.float32)] * 2,
  )


def kernel(x_user, x_item, edge_index_ui, edge_index_iu,
           Wl0_ui, bl0_ui, Wr0_ui, Wl0_iu, bl0_iu, Wr0_iu,
           g0_u, be0_u, g0_i, be0_i,
           Wl1_ui, bl1_ui, Wr1_ui, Wl1_iu, bl1_iu, Wr1_iu,
           g1_u, be1_u, g1_i, be1_i):
  n, d = x_user.shape
  e = edge_index_ui.shape[1]

  n_acc = -(-(n + 1) // (NS * CH)) * (NS * CH)
  n_src = n + 8
  e_pad = -(-e // (NS * CH * BLK)) * (NS * CH * BLK)
  nch = e_pad // (NS * CH)

  i32 = jnp.int32
  pad_idx = jnp.full((e_pad - e,), n, i32)
  r3 = lambda a: a.reshape(NS, nch, CH)
  s_ui = r3(jnp.concatenate([edge_index_ui[0].astype(i32), pad_idx]))
  d_ui = r3(jnp.concatenate([edge_index_ui[1].astype(i32), pad_idx]))
  s_iu = r3(jnp.concatenate([edge_index_iu[0].astype(i32), pad_idx]))
  d_iu = r3(jnp.concatenate([edge_index_iu[1].astype(i32), pad_idx]))

  zrow = jnp.zeros((n_src - n, d), jnp.float32)
  xu_pad = jnp.concatenate([x_user, zrow])
  xi_pad = jnp.concatenate([x_item, zrow])

  seg_c = _make_seg_kernel(n_acc, n_src, e_pad, d, with_counts=True)
  seg_n = _make_seg_kernel(n_acc, n_src, e_pad, d, with_counts=False)
  dense_pad = _make_dense_kernel(n, n_acc, d, n_src)
  dense_fin = _make_dense_kernel(n, n_acc, d, n)

  r2 = lambda v: v.reshape(1, d)

  sum_i0, sum_u0, cnt_i, cnt_u = seg_c(
      xu_pad, xi_pad, s_ui, d_ui, s_iu, d_iu)
  i1_pad, u1_pad = dense_pad(
      sum_i0, cnt_i, x_item, Wl0_ui, r2(bl0_ui), Wr0_ui, r2(g0_i), r2(be0_i),
      sum_u0, cnt_u, x_user, Wl0_iu, r2(bl0_iu), Wr0_iu, r2(g0_u), r2(be0_u))

  sum_i1, sum_u1 = seg_n(u1_pad, i1_pad, s_ui, d_ui, s_iu, d_iu)
  i2, u2 = dense_fin(
      sum_i1, cnt_i, i1_pad[0:n], Wl1_ui, r2(bl1_ui), Wr1_ui,
      r2(g1_i), r2(be1_i),
      sum_u1, cnt_u, u1_pad[0:n], Wl1_iu, r2(bl1_iu), Wr1_iu,
      r2(g1_u), r2(be1_u))

  return (x_user, x_item, u1_pad[0:n], i1_pad[0:n], u2, i2)


# speedup vs baseline: 1.1585x; 1.1488x over previous
import functools

import jax
import jax.numpy as jnp
from jax import lax
from jax.experimental import pallas as pl
from jax.experimental.pallas import tpu as pltpu
from jax.experimental.pallas import tpu_sc as plsc

NC = 2
NS = 16
CH = 128
BLK = 32


def _make_seg_kernel(n_acc, n_src_rows, e_pad, d, with_counts):
  rpt = n_acc // NS
  ept = e_pad // NS
  nch = ept // CH
  nblk = nch // BLK

  out_type = [jax.ShapeDtypeStruct((n_acc, d), jnp.float32)] * (
      4 if with_counts else 2)
  scratch = [
      pltpu.VMEM_SHARED((n_acc, d), jnp.float32),
      pltpu.VMEM((BLK + 1, CH), jnp.int32),
      pltpu.VMEM((BLK, CH), jnp.int32),
      pltpu.VMEM((2, CH, d), jnp.float32),
      pltpu.SemaphoreType.DMA,
      pltpu.SemaphoreType.DMA,
  ]

  mesh = plsc.VectorSubcoreMesh(core_axis_name="c", subcore_axis_name="s",
                                num_cores=NC, num_subcores=NS)

  def body(*refs):
    if with_counts:
      (x0, x1, si0, di0, si1, di1,
       sum0, sum1, cnt0, cnt1, acc, sall, dall, rows, g0, g1) = refs
    else:
      (x0, x1, si0, di0, si1, di1,
       sum0, sum1, acc, sall, dall, rows, g0, g1) = refs
    cid = lax.axis_index("c")
    sid = lax.axis_index("s")
    r0 = sid * rpt

    def fill(buf, value, dtype):
      v = jnp.full((16,), value, dtype)

      def fr(r, carry):
        for k in range(d // 16):
          buf[r, pl.ds(k * 16, 16)] = v
        return carry
      lax.fori_loop(0, CH, fr, 0)

    def zero_acc():
      fill(rows.at[1], 0.0, jnp.float32)
      for j in range(rpt // CH):
        pltpu.sync_copy(rows.at[1], acc.at[pl.ds(r0 + j * CH, CH)])

    def writeout(o_ref):
      for j in range(rpt // CH):
        pltpu.sync_copy(acc.at[pl.ds(r0 + j * CH, CH)], rows.at[0])
        pltpu.sync_copy(rows.at[0], o_ref.at[pl.ds(r0 + j * CH, CH)])

    def wait_gather(x_hbm, b, sem):
      pltpu.make_async_copy(x_hbm.at[sall.at[0]], rows.at[b], sem).wait()

    zero_acc()
    plsc.subcore_barrier()

    def do_edges(x_hbm, s3, d3):
      zvec = jnp.zeros((16,), jnp.int32)
      for k in range(CH // 16):
        sall[BLK, pl.ds(k * 16, 16)] = zvec

      def block(blk, carry):
        pltpu.sync_copy(s3.at[sid, pl.ds(blk * BLK, BLK)],
                        sall.at[pl.ds(0, BLK)])
        pltpu.sync_copy(d3.at[sid, pl.ds(blk * BLK, BLK)], dall)
        def step(c, carry2):
          pltpu.sync_copy(x_hbm.at[sall.at[c]], rows.at[0])
          pltpu.sync_copy(rows.at[0], acc.at[dall.at[c]], add=True)
          return carry2
        lax.fori_loop(0, BLK, step, 0)
        return carry
      lax.fori_loop(0, nblk, block, 0)

    pl.when(cid == 0)(lambda: do_edges(x0, si0, di0))
    pl.when(cid == 1)(lambda: do_edges(x1, si1, di1))
    plsc.subcore_barrier()
    pl.when(cid == 0)(lambda: writeout(sum0))
    pl.when(cid == 1)(lambda: writeout(sum1))

    if with_counts:
      zero_acc()
      fill(rows.at[0], 1.0, jnp.float32)
      plsc.subcore_barrier()

      def do_counts(d3):
        def cblock(blk, carry):
          pltpu.sync_copy(d3.at[sid, pl.ds(blk * BLK, BLK)], dall)

          def cstep(c, carry2):
            pltpu.sync_copy(rows.at[0], acc.at[dall.at[c]], add=True)
            return carry2
          lax.fori_loop(0, BLK, cstep, 0)
          return carry
        lax.fori_loop(0, nblk, cblock, 0)

      pl.when(cid == 0)(lambda: do_counts(di0))
      pl.when(cid == 1)(lambda: do_counts(di1))
      plsc.subcore_barrier()
      pl.when(cid == 0)(lambda: writeout(cnt0))
      pl.when(cid == 1)(lambda: writeout(cnt1))

  return pl.kernel(body, out_type=out_type, mesh=mesh, scratch_types=scratch)


def _make_dense_kernel(n, n_acc, d, out_rows):
  def one(s_ref, c_ref, x_ref, wl_ref, bl_ref, wr_ref, g_ref, be_ref, o_ref):
    cnt = jnp.maximum(c_ref[0:n, 0:1], 1.0)
    mean = s_ref[0:n, :] / cnt
    h = jnp.dot(mean, wl_ref[...], preferred_element_type=jnp.float32)
    h = h + bl_ref[...]
    h = h + jnp.dot(x_ref[...], wr_ref[...], preferred_element_type=jnp.float32)
    mu = jnp.mean(h, axis=0, keepdims=True)
    var = jnp.mean(jnp.square(h - mu), axis=0, keepdims=True)
    y = (h - mu) * lax.rsqrt(var + 1e-5) * g_ref[...] + be_ref[...]
    y = jnp.where(y > 0, y, jnp.exp(jnp.minimum(y, 0.0)) - 1.0)
    o_ref[0:n, :] = y
    if out_rows > n:
      o_ref[n:out_rows, :] = jnp.zeros((out_rows - n, d), jnp.float32)

  def body(s0, c0, x0, wl0, bl0, wr0, g0, be0,
           s1, c1, x1, wl1, bl1, wr1, g1, be1, o0, o1):
    one(s0, c0, x0, wl0, bl0, wr0, g0, be0, o0)
    one(s1, c1, x1, wl1, bl1, wr1, g1, be1, o1)

  return pl.pallas_call(
      body,
      out_shape=[jax.ShapeDtypeStruct((out_rows, d), jnp.float32)] * 2,
  )


def kernel(x_user, x_item, edge_index_ui, edge_index_iu,
           Wl0_ui, bl0_ui, Wr0_ui, Wl0_iu, bl0_iu, Wr0_iu,
           g0_u, be0_u, g0_i, be0_i,
           Wl1_ui, bl1_ui, Wr1_ui, Wl1_iu, bl1_iu, Wr1_iu,
           g1_u, be1_u, g1_i, be1_i):
  n, d = x_user.shape
  e = edge_index_ui.shape[1]

  n_acc = -(-(n + 1) // (NS * CH)) * (NS * CH)
  n_src = n + 8
  e_pad = -(-e // (NS * CH * BLK)) * (NS * CH * BLK)
  nch = e_pad // (NS * CH)

  i32 = jnp.int32
  pad_idx = jnp.full((e_pad - e,), n, i32)
  r3 = lambda a: a.reshape(NS, nch, CH)
  s_ui = r3(jnp.concatenate([edge_index_ui[0].astype(i32), pad_idx]))
  d_ui = r3(jnp.concatenate([edge_index_ui[1].astype(i32), pad_idx]))
  s_iu = r3(jnp.concatenate([edge_index_iu[0].astype(i32), pad_idx]))
  d_iu = r3(jnp.concatenate([edge_index_iu[1].astype(i32), pad_idx]))

  zrow = jnp.zeros((n_src - n, d), jnp.float32)
  xu_pad = jnp.concatenate([x_user, zrow])
  xi_pad = jnp.concatenate([x_item, zrow])

  seg_c = _make_seg_kernel(n_acc, n_src, e_pad, d, with_counts=True)
  seg_n = _make_seg_kernel(n_acc, n_src, e_pad, d, with_counts=False)
  dense_pad = _make_dense_kernel(n, n_acc, d, n_src)
  dense_fin = _make_dense_kernel(n, n_acc, d, n)

  r2 = lambda v: v.reshape(1, d)

  sum_i0, sum_u0, cnt_i, cnt_u = seg_c(
      xu_pad, xi_pad, s_ui, d_ui, s_iu, d_iu)
  i1_pad, u1_pad = dense_pad(
      sum_i0, cnt_i, x_item, Wl0_ui, r2(bl0_ui), Wr0_ui, r2(g0_i), r2(be0_i),
      sum_u0, cnt_u, x_user, Wl0_iu, r2(bl0_iu), Wr0_iu, r2(g0_u), r2(be0_u))

  sum_i1, sum_u1 = seg_n(u1_pad, i1_pad, s_ui, d_ui, s_iu, d_iu)
  i2, u2 = dense_fin(
      sum_i1, cnt_i, i1_pad[0:n], Wl1_ui, r2(bl1_ui), Wr1_ui,
      r2(g1_i), r2(be1_i),
      sum_u1, cnt_u, u1_pad[0:n], Wl1_iu, r2(bl1_iu), Wr1_iu,
      r2(g1_u), r2(be1_u))

  return (x_user, x_item, u1_pad[0:n], i1_pad[0:n], u2, i2)


# speedup vs baseline: 1.6699x; 1.4414x over previous
import functools

import jax
import jax.numpy as jnp
from jax import lax
from jax.experimental import pallas as pl
from jax.experimental.pallas import tpu as pltpu
from jax.experimental.pallas import tpu_sc as plsc

NC = 2
NS = 16
CH = 128


def _make_seg_kernel(n_acc, n_src_rows, e_pad, d, with_counts):
  rpt = n_acc // NS
  ept = e_pad // NS
  nch = ept // CH
  npair = nch // 2

  out_type = [jax.ShapeDtypeStruct((n_acc, d), jnp.float32)] * (
      4 if with_counts else 2)
  scratch = [
      pltpu.VMEM_SHARED((n_acc, d), jnp.float32),
      pltpu.VMEM((CH,), jnp.int32),
      pltpu.VMEM((CH,), jnp.int32),
      pltpu.VMEM((CH,), jnp.int32),
      pltpu.VMEM((2, CH, d), jnp.float32),
      pltpu.SemaphoreType.DMA,
      pltpu.SemaphoreType.DMA,
  ]

  mesh = plsc.VectorSubcoreMesh(core_axis_name="c", subcore_axis_name="s",
                                num_cores=NC, num_subcores=NS)

  def body(*refs):
    if with_counts:
      (x0, x1, s0, d0, s1, d1,
       sum0, sum1, cnt0, cnt1, acc, sidxa, sidxb, didx, rows, g0, g1) = refs
    else:
      (x0, x1, s0, d0, s1, d1,
       sum0, sum1, acc, sidxa, sidxb, didx, rows, g0, g1) = refs
    cid = lax.axis_index("c")
    sid = lax.axis_index("s")
    r0 = sid * rpt
    e0 = sid * ept

    def fill(buf, value):
      v = jnp.full((16,), value, jnp.float32)

      def fr(r, carry):
        for k in range(d // 16):
          buf[r, pl.ds(k * 16, 16)] = v
        return carry
      lax.fori_loop(0, CH, fr, 0)

    def zero_acc():
      fill(rows.at[1], 0.0)
      for j in range(rpt // CH):
        pltpu.sync_copy(rows.at[1], acc.at[pl.ds(r0 + j * CH, CH)])

    def writeout(o_ref):
      for j in range(rpt // CH):
        pltpu.sync_copy(acc.at[pl.ds(r0 + j * CH, CH)], rows.at[0])
        pltpu.sync_copy(rows.at[0], o_ref.at[pl.ds(r0 + j * CH, CH)])

    def wait_gather(x_hbm, b, sem):
      pltpu.make_async_copy(x_hbm.at[sidxa], rows.at[b], sem).wait()

    zero_acc()
    plsc.subcore_barrier()

    def do_edges(x_hbm, s_hbm, d_hbm):
      pltpu.sync_copy(s_hbm.at[pl.ds(e0, CH)], sidxa)
      pltpu.async_copy(x_hbm.at[sidxa], rows.at[0], g0)

      def pair(p, carry):
        b = e0 + 2 * p * CH
        pltpu.sync_copy(s_hbm.at[pl.ds(b + CH, CH)], sidxb)
        wait_gather(x_hbm, 0, g0)
        pltpu.async_copy(x_hbm.at[sidxb], rows.at[1], g1)
        pltpu.sync_copy(d_hbm.at[pl.ds(b, CH)], didx)
        pltpu.sync_copy(rows.at[0], acc.at[didx], add=True)
        pltpu.sync_copy(s_hbm.at[pl.ds(b + 2 * CH, CH)], sidxa)
        wait_gather(x_hbm, 1, g1)
        pltpu.async_copy(x_hbm.at[sidxa], rows.at[0], g0)
        pltpu.sync_copy(d_hbm.at[pl.ds(b + CH, CH)], didx)
        pltpu.sync_copy(rows.at[1], acc.at[didx], add=True)
        return carry
      lax.fori_loop(0, npair, pair, 0)
      wait_gather(x_hbm, 0, g0)

    pl.when(cid == 0)(lambda: do_edges(x0, s0, d0))
    pl.when(cid == 1)(lambda: do_edges(x1, s1, d1))
    plsc.subcore_barrier()
    pl.when(cid == 0)(lambda: writeout(sum0))
    pl.when(cid == 1)(lambda: writeout(sum1))

    if with_counts:
      zero_acc()
      fill(rows.at[0], 1.0)
      plsc.subcore_barrier()

      def do_counts(d_hbm):
        def cstep(c, carry):
          pltpu.sync_copy(d_hbm.at[pl.ds(e0 + c * CH, CH)], didx)
          pltpu.sync_copy(rows.at[0], acc.at[didx], add=True)
          return carry
        lax.fori_loop(0, nch, cstep, 0)

      pl.when(cid == 0)(lambda: do_counts(d0))
      pl.when(cid == 1)(lambda: do_counts(d1))
      plsc.subcore_barrier()
      pl.when(cid == 0)(lambda: writeout(cnt0))
      pl.when(cid == 1)(lambda: writeout(cnt1))

  return pl.kernel(body, out_type=out_type, mesh=mesh, scratch_types=scratch)


def _make_dense_kernel(n, n_acc, d, out_rows):
  def one(s_ref, c_ref, x_ref, wl_ref, bl_ref, wr_ref, g_ref, be_ref, o_ref):
    cnt = jnp.maximum(c_ref[0:n, 0:1], 1.0)
    mean = s_ref[0:n, :] / cnt
    h = jnp.dot(mean, wl_ref[...], preferred_element_type=jnp.float32)
    h = h + bl_ref[...]
    h = h + jnp.dot(x_ref[...], wr_ref[...], preferred_element_type=jnp.float32)
    mu = jnp.mean(h, axis=0, keepdims=True)
    var = jnp.mean(jnp.square(h - mu), axis=0, keepdims=True)
    y = (h - mu) * lax.rsqrt(var + 1e-5) * g_ref[...] + be_ref[...]
    y = jnp.where(y > 0, y, jnp.exp(jnp.minimum(y, 0.0)) - 1.0)
    o_ref[0:n, :] = y
    if out_rows > n:
      o_ref[n:out_rows, :] = jnp.zeros((out_rows - n, d), jnp.float32)

  def body(s0, c0, x0, wl0, bl0, wr0, g0, be0,
           s1, c1, x1, wl1, bl1, wr1, g1, be1, o0, o1):
    one(s0, c0, x0, wl0, bl0, wr0, g0, be0, o0)
    one(s1, c1, x1, wl1, bl1, wr1, g1, be1, o1)

  return pl.pallas_call(
      body,
      out_shape=[jax.ShapeDtypeStruct((out_rows, d), jnp.float32)] * 2,
  )


def kernel(x_user, x_item, edge_index_ui, edge_index_iu,
           Wl0_ui, bl0_ui, Wr0_ui, Wl0_iu, bl0_iu, Wr0_iu,
           g0_u, be0_u, g0_i, be0_i,
           Wl1_ui, bl1_ui, Wr1_ui, Wl1_iu, bl1_iu, Wr1_iu,
           g1_u, be1_u, g1_i, be1_i):
  n, d = x_user.shape
  e = edge_index_ui.shape[1]

  n_acc = -(-(n + 1) // (NS * CH)) * (NS * CH)
  n_src = n + 8
  e_pad = -(-e // (NS * CH * 2)) * (NS * CH * 2)

  i32 = jnp.int32
  pad_s = jnp.full((e_pad + CH - e,), n, i32)
  pad_d = jnp.full((e_pad - e,), n, i32)
  s_ui = jnp.concatenate([edge_index_ui[0].astype(i32), pad_s])
  d_ui = jnp.concatenate([edge_index_ui[1].astype(i32), pad_d])
  s_iu = jnp.concatenate([edge_index_iu[0].astype(i32), pad_s])
  d_iu = jnp.concatenate([edge_index_iu[1].astype(i32), pad_d])

  zrow = jnp.zeros((n_src - n, d), jnp.float32)
  xu_pad = jnp.concatenate([x_user, zrow])
  xi_pad = jnp.concatenate([x_item, zrow])

  seg_c = _make_seg_kernel(n_acc, n_src, e_pad, d, with_counts=True)
  seg_n = _make_seg_kernel(n_acc, n_src, e_pad, d, with_counts=False)
  dense_pad = _make_dense_kernel(n, n_acc, d, n_src)
  dense_fin = _make_dense_kernel(n, n_acc, d, n)

  r2 = lambda v: v.reshape(1, d)

  sum_i0, sum_u0, cnt_i, cnt_u = seg_c(
      xu_pad, xi_pad, s_ui, d_ui, s_iu, d_iu)
  i1_pad, u1_pad = dense_pad(
      sum_i0, cnt_i, x_item, Wl0_ui, r2(bl0_ui), Wr0_ui, r2(g0_i), r2(be0_i),
      sum_u0, cnt_u, x_user, Wl0_iu, r2(bl0_iu), Wr0_iu, r2(g0_u), r2(be0_u))

  sum_i1, sum_u1 = seg_n(u1_pad, i1_pad, s_ui, d_ui, s_iu, d_iu)
  i2, u2 = dense_fin(
      sum_i1, cnt_i, i1_pad[0:n], Wl1_ui, r2(bl1_ui), Wr1_ui,
      r2(g1_i), r2(be1_i),
      sum_u1, cnt_u, u1_pad[0:n], Wl1_iu, r2(bl1_iu), Wr1_iu,
      r2(g1_u), r2(be1_u))

  return (x_user, x_item, u1_pad[0:n], i1_pad[0:n], u2, i2)


# speedup vs baseline: 1.9602x; 1.1738x over previous
import functools

import jax
import jax.numpy as jnp
from jax import lax
from jax.experimental import pallas as pl
from jax.experimental.pallas import tpu as pltpu
from jax.experimental.pallas import tpu_sc as plsc

NC = 2
NS = 16
CH = 128


def _make_seg_kernel(n_acc, n_src_rows, e_pad, d, with_counts):
  rpt = n_acc // NS
  ept = e_pad // NS
  nch = ept // CH
  npair = nch // 2
  cpt = n_acc // NS

  out_type = [jax.ShapeDtypeStruct((n_acc, d), jnp.float32)] * 2
  scratch = [
      pltpu.VMEM_SHARED((n_acc, d), jnp.float32),
      pltpu.VMEM((CH,), jnp.int32),
      pltpu.VMEM((CH,), jnp.int32),
      pltpu.VMEM((CH,), jnp.int32),
      pltpu.VMEM((2, CH, d), jnp.float32),
      pltpu.SemaphoreType.DMA,
      pltpu.SemaphoreType.DMA,
  ]
  if with_counts:
    out_type += [jax.ShapeDtypeStruct((NS, n_acc), jnp.float32)] * 2
    scratch += [
        pltpu.VMEM((n_acc,), jnp.float32),
    ]

  mesh = plsc.VectorSubcoreMesh(core_axis_name="c", subcore_axis_name="s",
                                num_cores=NC, num_subcores=NS)

  def body(*refs):
    if with_counts:
      (x0, x1, s0, d0, s1, d1,
       sum0, sum1, cnt0, cnt1, acc, sidxa, sidxb, didx, rows, g0, g1,
       cntl) = refs
    else:
      (x0, x1, s0, d0, s1, d1,
       sum0, sum1, acc, sidxa, sidxb, didx, rows, g0, g1) = refs
    cid = lax.axis_index("c")
    sid = lax.axis_index("s")
    r0 = sid * rpt
    e0 = sid * ept
    ones16 = jnp.ones((16,), jnp.float32)
    zero16 = jnp.zeros((16,), jnp.float32)

    def fill(buf, value):
      v = jnp.full((16,), value, jnp.float32)

      def fr(r, carry):
        for k in range(d // 16):
          buf[r, pl.ds(k * 16, 16)] = v
        return carry
      lax.fori_loop(0, CH, fr, 0)

    def zero_acc():
      fill(rows.at[1], 0.0)
      for j in range(rpt // CH):
        pltpu.sync_copy(rows.at[1], acc.at[pl.ds(r0 + j * CH, CH)])

    def writeout(o_ref):
      for j in range(rpt // CH):
        pltpu.sync_copy(acc.at[pl.ds(r0 + j * CH, CH)], rows.at[0])
        pltpu.sync_copy(rows.at[0], o_ref.at[pl.ds(r0 + j * CH, CH)])

    def wait_gather(x_hbm, b, sem):
      pltpu.make_async_copy(x_hbm.at[sidxa], rows.at[b], sem).wait()

    zero_acc()
    if with_counts:
      def zc(i, carry):
        cntl[pl.ds(i * 16, 16)] = zero16
        return carry
      lax.fori_loop(0, n_acc // 16, zc, 0)
    plsc.subcore_barrier()

    def count_chunk():
      for k in range(CH // 16):
        iv = didx[pl.ds(k * 16, 16)]
        plsc.addupdate_scatter(cntl, [iv], ones16)

    def do_edges(x_hbm, s_hbm, d_hbm):
      pltpu.sync_copy(s_hbm.at[pl.ds(e0, CH)], sidxa)
      pltpu.async_copy(x_hbm.at[sidxa], rows.at[0], g0)

      def pair(p, carry):
        b = e0 + 2 * p * CH
        pltpu.sync_copy(s_hbm.at[pl.ds(b + CH, CH)], sidxb)
        wait_gather(x_hbm, 0, g0)
        pltpu.async_copy(x_hbm.at[sidxb], rows.at[1], g1)
        pltpu.sync_copy(d_hbm.at[pl.ds(b, CH)], didx)
        pltpu.sync_copy(rows.at[0], acc.at[didx], add=True)
        if with_counts:
          count_chunk()
        pltpu.sync_copy(s_hbm.at[pl.ds(b + 2 * CH, CH)], sidxa)
        wait_gather(x_hbm, 1, g1)
        pltpu.async_copy(x_hbm.at[sidxa], rows.at[0], g0)
        pltpu.sync_copy(d_hbm.at[pl.ds(b + CH, CH)], didx)
        pltpu.sync_copy(rows.at[1], acc.at[didx], add=True)
        if with_counts:
          count_chunk()
        return carry
      lax.fori_loop(0, npair, pair, 0)
      wait_gather(x_hbm, 0, g0)

    pl.when(cid == 0)(lambda: do_edges(x0, s0, d0))
    pl.when(cid == 1)(lambda: do_edges(x1, s1, d1))
    plsc.subcore_barrier()
    pl.when(cid == 0)(lambda: writeout(sum0))
    pl.when(cid == 1)(lambda: writeout(sum1))

    if with_counts:
      pl.when(cid == 0)(lambda: pltpu.sync_copy(cntl, cnt0.at[sid]))
      pl.when(cid == 1)(lambda: pltpu.sync_copy(cntl, cnt1.at[sid]))

  return pl.kernel(
      body, out_type=out_type, mesh=mesh, scratch_types=scratch,
      compiler_params=pltpu.CompilerParams(needs_layout_passes=False))


def _make_dense_kernel(n, n_acc, d, out_rows):
  def one(s_ref, c_ref, x_ref, wl_ref, bl_ref, wr_ref, g_ref, be_ref, o_ref):
    cnt = jnp.reshape(jnp.sum(c_ref[...], axis=0), (n_acc, 1))[0:n]
    cnt = jnp.maximum(cnt, 1.0)
    mean = s_ref[0:n, :] / cnt
    h = jnp.dot(mean, wl_ref[...], preferred_element_type=jnp.float32)
    h = h + bl_ref[...]
    h = h + jnp.dot(x_ref[...], wr_ref[...], preferred_element_type=jnp.float32)
    mu = jnp.mean(h, axis=0, keepdims=True)
    var = jnp.mean(jnp.square(h - mu), axis=0, keepdims=True)
    y = (h - mu) * lax.rsqrt(var + 1e-5) * g_ref[...] + be_ref[...]
    y = jnp.where(y > 0, y, jnp.exp(jnp.minimum(y, 0.0)) - 1.0)
    o_ref[0:n, :] = y
    if out_rows > n:
      o_ref[n:out_rows, :] = jnp.zeros((out_rows - n, d), jnp.float32)

  def body(s0, c0, x0, wl0, bl0, wr0, g0, be0,
           s1, c1, x1, wl1, bl1, wr1, g1, be1, o0, o1):
    one(s0, c0, x0, wl0, bl0, wr0, g0, be0, o0)
    one(s1, c1, x1, wl1, bl1, wr1, g1, be1, o1)

  return pl.pallas_call(
      body,
      out_shape=[jax.ShapeDtypeStruct((out_rows, d), jnp.float32)] * 2,
  )


def kernel(x_user, x_item, edge_index_ui, edge_index_iu,
           Wl0_ui, bl0_ui, Wr0_ui, Wl0_iu, bl0_iu, Wr0_iu,
           g0_u, be0_u, g0_i, be0_i,
           Wl1_ui, bl1_ui, Wr1_ui, Wl1_iu, bl1_iu, Wr1_iu,
           g1_u, be1_u, g1_i, be1_i):
  n, d = x_user.shape
  e = edge_index_ui.shape[1]

  n_acc = -(-(n + 1) // (NS * CH)) * (NS * CH)
  n_src = n + 8
  e_pad = -(-e // (NS * CH * 2)) * (NS * CH * 2)

  i32 = jnp.int32
  pad_s = jnp.full((e_pad + CH - e,), n, i32)
  pad_d = jnp.full((e_pad - e,), n, i32)
  s_ui = jnp.concatenate([edge_index_ui[0].astype(i32), pad_s])
  d_ui = jnp.concatenate([edge_index_ui[1].astype(i32), pad_d])
  s_iu = jnp.concatenate([edge_index_iu[0].astype(i32), pad_s])
  d_iu = jnp.concatenate([edge_index_iu[1].astype(i32), pad_d])

  zrow = jnp.zeros((n_src - n, d), jnp.float32)
  xu_pad = jnp.concatenate([x_user, zrow])
  xi_pad = jnp.concatenate([x_item, zrow])

  seg_c = _make_seg_kernel(n_acc, n_src, e_pad, d, with_counts=True)
  seg_n = _make_seg_kernel(n_acc, n_src, e_pad, d, with_counts=False)
  dense_pad = _make_dense_kernel(n, n_acc, d, n_src)
  dense_fin = _make_dense_kernel(n, n_acc, d, n)

  r2 = lambda v: v.reshape(1, d)

  sum_i0, sum_u0, cnt_i, cnt_u = seg_c(
      xu_pad, xi_pad, s_ui, d_ui, s_iu, d_iu)
  i1_pad, u1_pad = dense_pad(
      sum_i0, cnt_i, x_item, Wl0_ui, r2(bl0_ui), Wr0_ui, r2(g0_i), r2(be0_i),
      sum_u0, cnt_u, x_user, Wl0_iu, r2(bl0_iu), Wr0_iu, r2(g0_u), r2(be0_u))

  sum_i1, sum_u1 = seg_n(u1_pad, i1_pad, s_ui, d_ui, s_iu, d_iu)
  i2, u2 = dense_fin(
      sum_i1, cnt_i, i1_pad[0:n], Wl1_ui, r2(bl1_ui), Wr1_ui,
      r2(g1_i), r2(be1_i),
      sum_u1, cnt_u, u1_pad[0:n], Wl1_iu, r2(bl1_iu), Wr1_iu,
      r2(g1_u), r2(be1_u))

  return (x_user, x_item, u1_pad[0:n], i1_pad[0:n], u2, i2)


# speedup vs baseline: 2.0839x; 1.0631x over previous
import functools

import jax
import jax.numpy as jnp
from jax import lax
from jax.experimental import pallas as pl
from jax.experimental.pallas import tpu as pltpu
from jax.experimental.pallas import tpu_sc as plsc

NC = 2
NS = 16
CH = 128


def _make_seg_kernel(n_acc, n_src_rows, e_pad, d, with_counts):
  rpt = n_acc // NS
  ept = e_pad // NS
  nch = ept // CH
  npair = nch // 2
  cpt = n_acc // NS

  out_type = [jax.ShapeDtypeStruct((n_acc, d), jnp.float32)] * 2
  scratch = [
      pltpu.VMEM_SHARED((n_acc, d), jnp.float32),
      pltpu.VMEM((CH,), jnp.int32),
      pltpu.VMEM((CH,), jnp.int32),
      pltpu.VMEM((CH,), jnp.int32),
      pltpu.VMEM((CH,), jnp.int32),
      pltpu.VMEM((2, CH, d), jnp.float32),
      pltpu.SemaphoreType.DMA,
      pltpu.SemaphoreType.DMA,
      pltpu.SemaphoreType.DMA,
      pltpu.SemaphoreType.DMA,
      pltpu.SemaphoreType.DMA,
      pltpu.SemaphoreType.DMA,
  ]
  if with_counts:
    out_type += [jax.ShapeDtypeStruct((NS, n_acc), jnp.float32)] * 2
    scratch += [
        pltpu.VMEM((n_acc,), jnp.float32),
    ]

  mesh = plsc.VectorSubcoreMesh(core_axis_name="c", subcore_axis_name="s",
                                num_cores=NC, num_subcores=NS)

  def body(*refs):
    if with_counts:
      (x0, x1, s0, d0, s1, d1,
       sum0, sum1, cnt0, cnt1, acc, sidxa, sidxb, didxa, didxb, rows,
       g0, g1, is0, is1, id0, id1, cntl) = refs
    else:
      (x0, x1, s0, d0, s1, d1,
       sum0, sum1, acc, sidxa, sidxb, didxa, didxb, rows,
       g0, g1, is0, is1, id0, id1) = refs
    cid = lax.axis_index("c")
    sid = lax.axis_index("s")
    r0 = sid * rpt
    e0 = sid * ept
    ones16 = jnp.ones((16,), jnp.float32)
    zero16 = jnp.zeros((16,), jnp.float32)

    def fill(buf, value):
      v = jnp.full((16,), value, jnp.float32)

      def fr(r, carry):
        for k in range(d // 16):
          buf[r, pl.ds(k * 16, 16)] = v
        return carry
      lax.fori_loop(0, CH, fr, 0)

    def zero_acc():
      fill(rows.at[1], 0.0)
      for j in range(rpt // CH):
        pltpu.sync_copy(rows.at[1], acc.at[pl.ds(r0 + j * CH, CH)])

    def writeout(o_ref):
      for j in range(rpt // CH):
        pltpu.sync_copy(acc.at[pl.ds(r0 + j * CH, CH)], rows.at[0])
        pltpu.sync_copy(rows.at[0], o_ref.at[pl.ds(r0 + j * CH, CH)])

    def wait_gather(x_hbm, b, sem):
      pltpu.make_async_copy(x_hbm.at[sidxa], rows.at[b], sem).wait()

    zero_acc()
    if with_counts:
      def zc(i, carry):
        cntl[pl.ds(i * 16, 16)] = zero16
        return carry
      lax.fori_loop(0, n_acc // 16, zc, 0)
    plsc.subcore_barrier()

    def count_chunk(didx):
      for k in range(CH // 16):
        iv = didx[pl.ds(k * 16, 16)]
        plsc.addupdate_scatter(cntl, [iv], ones16)

    def wait_idx(i_hbm, buf, sem):
      pltpu.make_async_copy(i_hbm.at[pl.ds(0, CH)], buf, sem).wait()

    def do_edges(x_hbm, s_hbm, d_hbm):
      pltpu.sync_copy(s_hbm.at[pl.ds(e0, CH)], sidxa)
      pltpu.async_copy(x_hbm.at[sidxa], rows.at[0], g0)
      pltpu.async_copy(s_hbm.at[pl.ds(e0 + CH, CH)], sidxb, is1)
      pltpu.async_copy(d_hbm.at[pl.ds(e0, CH)], didxa, id0)

      def half(b, sa, sb, da, db, ga, gb, isb, isa2, ida, idb):
        wait_idx(s_hbm, sb, isb)
        wait_gather(x_hbm, 0 if ga is g0 else 1, ga)
        pltpu.async_copy(x_hbm.at[sb], rows.at[1 if ga is g0 else 0], gb)
        pltpu.async_copy(s_hbm.at[pl.ds(b + 2 * CH, CH)], sa, isa2)
        wait_idx(d_hbm, da, ida)
        pltpu.async_copy(d_hbm.at[pl.ds(b + CH, CH)], db, idb)
        pltpu.sync_copy(rows.at[0 if ga is g0 else 1], acc.at[da], add=True)
        if with_counts:
          count_chunk(da)

      def pair(p, carry):
        b = e0 + 2 * p * CH
        half(b, sidxa, sidxb, didxa, didxb, g0, g1, is1, is0, id0, id1)
        half(b + CH, sidxb, sidxa, didxb, didxa, g1, g0, is0, is1, id1, id0)
        return carry
      lax.fori_loop(0, npair, pair, 0)
      wait_gather(x_hbm, 0, g0)
      wait_idx(s_hbm, sidxb, is1)
      wait_idx(d_hbm, didxa, id0)

    pl.when(cid == 0)(lambda: do_edges(x0, s0, d0))
    pl.when(cid == 1)(lambda: do_edges(x1, s1, d1))
    plsc.subcore_barrier()
    pl.when(cid == 0)(lambda: writeout(sum0))
    pl.when(cid == 1)(lambda: writeout(sum1))

    if with_counts:
      pl.when(cid == 0)(lambda: pltpu.sync_copy(cntl, cnt0.at[sid]))
      pl.when(cid == 1)(lambda: pltpu.sync_copy(cntl, cnt1.at[sid]))

  return pl.kernel(
      body, out_type=out_type, mesh=mesh, scratch_types=scratch,
      compiler_params=pltpu.CompilerParams(needs_layout_passes=False))


def _make_dense_kernel(n, n_acc, d, out_rows):
  def one(s_ref, c_ref, x_ref, wl_ref, bl_ref, wr_ref, g_ref, be_ref, o_ref):
    cnt = jnp.reshape(jnp.sum(c_ref[...], axis=0), (n_acc, 1))[0:n]
    cnt = jnp.maximum(cnt, 1.0)
    mean = s_ref[0:n, :] / cnt
    h = jnp.dot(mean, wl_ref[...], preferred_element_type=jnp.float32)
    h = h + bl_ref[...]
    h = h + jnp.dot(x_ref[...], wr_ref[...], preferred_element_type=jnp.float32)
    mu = jnp.mean(h, axis=0, keepdims=True)
    var = jnp.mean(jnp.square(h - mu), axis=0, keepdims=True)
    y = (h - mu) * lax.rsqrt(var + 1e-5) * g_ref[...] + be_ref[...]
    y = jnp.where(y > 0, y, jnp.exp(jnp.minimum(y, 0.0)) - 1.0)
    o_ref[0:n, :] = y
    if out_rows > n:
      o_ref[n:out_rows, :] = jnp.zeros((out_rows - n, d), jnp.float32)

  def body(s0, c0, x0, wl0, bl0, wr0, g0, be0,
           s1, c1, x1, wl1, bl1, wr1, g1, be1, o0, o1):
    one(s0, c0, x0, wl0, bl0, wr0, g0, be0, o0)
    one(s1, c1, x1, wl1, bl1, wr1, g1, be1, o1)

  return pl.pallas_call(
      body,
      out_shape=[jax.ShapeDtypeStruct((out_rows, d), jnp.float32)] * 2,
  )


def kernel(x_user, x_item, edge_index_ui, edge_index_iu,
           Wl0_ui, bl0_ui, Wr0_ui, Wl0_iu, bl0_iu, Wr0_iu,
           g0_u, be0_u, g0_i, be0_i,
           Wl1_ui, bl1_ui, Wr1_ui, Wl1_iu, bl1_iu, Wr1_iu,
           g1_u, be1_u, g1_i, be1_i):
  n, d = x_user.shape
  e = edge_index_ui.shape[1]

  n_acc = -(-(n + 1) // (NS * CH)) * (NS * CH)
  n_src = n + 8
  e_pad = -(-e // (NS * CH * 2)) * (NS * CH * 2)

  i32 = jnp.int32
  pad_s = jnp.full((e_pad + 2 * CH - e,), n, i32)
  pad_d = jnp.full((e_pad + CH - e,), n, i32)
  s_ui = jnp.concatenate([edge_index_ui[0].astype(i32), pad_s])
  d_ui = jnp.concatenate([edge_index_ui[1].astype(i32), pad_d])
  s_iu = jnp.concatenate([edge_index_iu[0].astype(i32), pad_s])
  d_iu = jnp.concatenate([edge_index_iu[1].astype(i32), pad_d])

  zrow = jnp.zeros((n_src - n, d), jnp.float32)
  xu_pad = jnp.concatenate([x_user, zrow])
  xi_pad = jnp.concatenate([x_item, zrow])

  seg_c = _make_seg_kernel(n_acc, n_src, e_pad, d, with_counts=True)
  seg_n = _make_seg_kernel(n_acc, n_src, e_pad, d, with_counts=False)
  dense_pad = _make_dense_kernel(n, n_acc, d, n_src)
  dense_fin = _make_dense_kernel(n, n_acc, d, n)

  r2 = lambda v: v.reshape(1, d)

  sum_i0, sum_u0, cnt_i, cnt_u = seg_c(
      xu_pad, xi_pad, s_ui, d_ui, s_iu, d_iu)
  i1_pad, u1_pad = dense_pad(
      sum_i0, cnt_i, x_item, Wl0_ui, r2(bl0_ui), Wr0_ui, r2(g0_i), r2(be0_i),
      sum_u0, cnt_u, x_user, Wl0_iu, r2(bl0_iu), Wr0_iu, r2(g0_u), r2(be0_u))

  sum_i1, sum_u1 = seg_n(u1_pad, i1_pad, s_ui, d_ui, s_iu, d_iu)
  i2, u2 = dense_fin(
      sum_i1, cnt_i, i1_pad[0:n], Wl1_ui, r2(bl1_ui), Wr1_ui,
      r2(g1_i), r2(be1_i),
      sum_u1, cnt_u, u1_pad[0:n], Wl1_iu, r2(bl1_iu), Wr1_iu,
      r2(g1_u), r2(be1_u))

  return (x_user, x_item, u1_pad[0:n], i1_pad[0:n], u2, i2)
